# baseline (speedup 1.0000x reference)
import functools

import jax
import jax.numpy as jnp
from jax import lax
from jax.experimental import pallas as pl
from jax.experimental.pallas import tpu as pltpu
from jax.experimental.pallas import tpu_sc as plsc


def _sc_gather_pairs(ids, role, table, V, D):
    N = ids.shape[0]
    info = plsc.get_sparse_core_info()
    NC, NS = info.num_cores, info.num_subcores
    NW = NC * NS
    per_w = N // NW
    C = 512
    nch = per_w // C
    assert per_w % C == 0 and N % NW == 0

    mesh = plsc.VectorSubcoreMesh(core_axis_name="c", subcore_axis_name="s")

    @functools.partial(
        pl.kernel,
        mesh=mesh,
        out_type=jax.ShapeDtypeStruct((2 * N, D), jnp.float32),
        scratch_types=[
            pltpu.VMEM((per_w,), jnp.int32),
            pltpu.VMEM((per_w,), jnp.int32),
            pltpu.VMEM((2 * C,), jnp.int32),
            pltpu.VMEM((2 * C, D), jnp.float32),
            pltpu.SemaphoreType.DMA,
        ],
        compiler_params=pltpu.CompilerParams(
            use_tc_tiling_on_sc=False, needs_layout_passes=False
        ),
    )
    def gather_kernel(ids_hbm, role_hbm, tbl_hbm, out_hbm,
                      ids_v, role_v, idx2_v, rows2_v, sem):
        wid = lax.axis_index("s") * NC + lax.axis_index("c")
        base = wid * per_w
        pltpu.sync_copy(ids_hbm.at[pl.ds(base, per_w)], ids_v)
        pltpu.sync_copy(role_hbm.at[pl.ds(base, per_w)], role_v)
        tbl = tbl_hbm

        def chunk(i, carry):
            choff = i * C

            def build(k, _):
                sl = pl.ds(choff + k * 16, 16)
                idv = ids_v[sl]
                rv = role_v[sl]
                b = 32 * k + 2 * lax.iota(jnp.int32, 16)
                plsc.store_scatter(idx2_v, [b + rv], idv)
                plsc.store_scatter(idx2_v, [b + (1 - rv)],
                                   jnp.zeros((16,), jnp.int32))
                return _

            lax.fori_loop(0, C // 16, build, 0)
            pltpu.async_copy(tbl.at[idx2_v], rows2_v, sem).wait()
            pltpu.sync_copy(
                rows2_v, out_hbm.at[pl.ds(2 * (base + choff), 2 * C)]
            )
            return carry

        lax.fori_loop(0, nch, chunk, 0)

    return gather_kernel(ids, role, table)


def _tc_matmul(xa, wstack, blk):
    N = xa.shape[0]
    D = wstack.shape[1]

    def body(x_ref, w_ref, o_ref):
        o_ref[...] = jnp.dot(
            x_ref[...], w_ref[...], preferred_element_type=jnp.float32
        )

    return pl.pallas_call(
        body,
        grid=(N // blk,),
        in_specs=[
            pl.BlockSpec((blk, 2 * D), lambda i: (i, 0)),
            pl.BlockSpec((2 * D, D), lambda i: (0, 0)),
        ],
        out_specs=pl.BlockSpec((blk, D), lambda i: (i, 0)),
        out_shape=jax.ShapeDtypeStruct((N, D), jnp.float32),
    )(xa, wstack)


def kernel(input_ids, role_mask, table, W0, W1):
    B, L = input_ids.shape
    V, D = table.shape
    N = B * L
    ids = input_ids.reshape(N).astype(jnp.int32)
    role = role_mask.reshape(N).astype(jnp.int32)
    xa2 = _sc_gather_pairs(ids, role, table, V, D)
    xa = xa2.reshape(N, 2 * D)
    wstack = jnp.concatenate([W0.T, W1.T], axis=0)
    out = _tc_matmul(xa, wstack, blk=4096)
    return out.reshape(B, L, D)


# speedup vs baseline: 11.0674x; 11.0674x over previous
import functools

import jax
import jax.numpy as jnp
from jax import lax
from jax.experimental import pallas as pl
from jax.experimental.pallas import tpu as pltpu
from jax.experimental.pallas import tpu_sc as plsc


def _sc_gather_pairs(ids, role, table, V, D):
    N = ids.shape[0]
    info = plsc.get_sparse_core_info()
    NC, NS = info.num_cores, info.num_subcores
    NW = NC * NS
    per_w = N // NW
    C = 256
    nch = per_w // C
    assert per_w % C == 0 and N % NW == 0

    mesh = plsc.VectorSubcoreMesh(core_axis_name="c", subcore_axis_name="s")

    @functools.partial(
        pl.kernel,
        mesh=mesh,
        out_type=jax.ShapeDtypeStruct((2 * N, D), jnp.float32),
        scratch_types=[
            pltpu.VMEM((C,), jnp.int32),
            pltpu.VMEM((C,), jnp.int32),
            pltpu.VMEM((C,), jnp.int32),
            pltpu.VMEM((C,), jnp.int32),
            pltpu.VMEM((C, D), jnp.float32),
            pltpu.VMEM((C, D), jnp.float32),
            pltpu.VMEM_SHARED((NS * 2 * C, D), jnp.float32),
            pltpu.SemaphoreType.DMA,
        ],
        compiler_params=pltpu.CompilerParams(
            use_tc_tiling_on_sc=False, needs_layout_passes=False
        ),
    )
    def gather_kernel(ids_hbm, role_hbm, tbl_hbm, out_hbm,
                      idc_v, rol_v, dst_v, dstz_v, zeros_v, rows_v,
                      rows2_v, sem):
        sid = lax.axis_index("s")
        wid = sid * NC + lax.axis_index("c")
        base = wid * per_w
        sbase = sid * 2 * C

        def zinit(j, _):
            z = jnp.zeros((16,), jnp.float32)
            zeros_v[j, pl.ds(0, 16)] = z
            zeros_v[j, pl.ds(16, 16)] = z
            zeros_v[j, pl.ds(32, 16)] = z
            zeros_v[j, pl.ds(48, 16)] = z
            return _

        lax.fori_loop(0, C, zinit, 0)

        def chunk(i, carry):
            choff = base + i * C
            pltpu.sync_copy(ids_hbm.at[pl.ds(choff, C)], idc_v)
            pltpu.sync_copy(role_hbm.at[pl.ds(choff, C)], rol_v)

            def build(k, _):
                sl = pl.ds(k * 16, 16)
                rv = rol_v[sl]
                b = sbase + 32 * k + 2 * lax.iota(jnp.int32, 16)
                dst_v[sl] = b + rv
                dstz_v[sl] = b + (1 - rv)
                return _

            lax.fori_loop(0, C // 16, build, 0)
            pltpu.async_copy(tbl_hbm.at[idc_v], rows_v, sem).wait()
            pltpu.async_copy(zeros_v, rows2_v.at[dstz_v], sem).wait()
            pltpu.async_copy(rows_v, rows2_v.at[dst_v], sem).wait()
            pltpu.sync_copy(rows2_v.at[pl.ds(sbase, 2 * C)],
                            out_hbm.at[pl.ds(2 * choff, 2 * C)])
            return carry

        lax.fori_loop(0, nch, chunk, 0)

    return gather_kernel(ids, role, table)


def _tc_matmul(xa, wstack, blk):
    N = xa.shape[0]
    D = wstack.shape[1]

    def body(x_ref, w_ref, o_ref):
        o_ref[...] = jnp.dot(
            x_ref[...], w_ref[...], preferred_element_type=jnp.float32
        )

    return pl.pallas_call(
        body,
        grid=(N // blk,),
        in_specs=[
            pl.BlockSpec((blk, 2 * D), lambda i: (i, 0)),
            pl.BlockSpec((2 * D, D), lambda i: (0, 0)),
        ],
        out_specs=pl.BlockSpec((blk, D), lambda i: (i, 0)),
        out_shape=jax.ShapeDtypeStruct((N, D), jnp.float32),
    )(xa, wstack)


def kernel(input_ids, role_mask, table, W0, W1):
    B, L = input_ids.shape
    V, D = table.shape
    N = B * L
    ids = input_ids.reshape(N).astype(jnp.int32)
    role = role_mask.reshape(N).astype(jnp.int32)
    xa2 = _sc_gather_pairs(ids, role, table, V, D)
    xa = xa2.reshape(N, 2 * D)
    wstack = jnp.concatenate([W0.T, W1.T], axis=0)
    out = _tc_matmul(xa, wstack, blk=4096)
    return out.reshape(B, L, D)


# speedup vs baseline: 13.0668x; 1.1807x over previous
import functools

import jax
import jax.numpy as jnp
from jax import lax
from jax.experimental import pallas as pl
from jax.experimental.pallas import tpu as pltpu
from jax.experimental.pallas import tpu_sc as plsc


def _sc_gather_pairs(ids, role, table, V, D):
    N = ids.shape[0]
    info = plsc.get_sparse_core_info()
    NC, NS = info.num_cores, info.num_subcores
    NW = NC * NS
    per_w = N // NW
    C = 320
    nch = per_w // C
    assert per_w % C == 0 and N % NW == 0 and nch % 2 == 0

    mesh = plsc.VectorSubcoreMesh(core_axis_name="c", subcore_axis_name="s")

    @functools.partial(
        pl.kernel,
        mesh=mesh,
        out_type=jax.ShapeDtypeStruct((2 * N, D), jnp.float32),
        scratch_types=[
            pltpu.VMEM((per_w,), jnp.int32),
            pltpu.VMEM((per_w,), jnp.int32),
            pltpu.VMEM((2, C), jnp.int32),
            pltpu.VMEM((2, C), jnp.int32),
            pltpu.VMEM((C, D), jnp.float32),
            pltpu.VMEM((2, C, D), jnp.float32),
            pltpu.SemaphoreType.DMA,
            pltpu.SemaphoreType.DMA,
        ],
        compiler_params=pltpu.CompilerParams(
            use_tc_tiling_on_sc=False, needs_layout_passes=False
        ),
    )
    def gather_kernel(ids_hbm, role_hbm, tbl_hbm, out_hbm,
                      ids_v, rol_v, dst_v, dstz_v, zeros_v, rows_v,
                      sem_g, sem_s):
        sid = lax.axis_index("s")
        wid = sid * NC + lax.axis_index("c")
        base = wid * per_w

        pltpu.sync_copy(ids_hbm.at[pl.ds(base, per_w)], ids_v)
        pltpu.sync_copy(role_hbm.at[pl.ds(base, per_w)], rol_v)

        def zinit(j, _):
            z = jnp.zeros((16,), jnp.float32)
            zeros_v[j, pl.ds(0, 16)] = z
            zeros_v[j, pl.ds(16, 16)] = z
            zeros_v[j, pl.ds(32, 16)] = z
            zeros_v[j, pl.ds(48, 16)] = z
            return _

        lax.fori_loop(0, C, zinit, 0)

        def gather_of(i, b):
            return pltpu.make_async_copy(
                tbl_hbm.at[ids_v.at[pl.ds(i * C, C)]],
                rows_v.at[b], sem_g,
            )

        def scatters_of(i, b):
            return (
                pltpu.make_async_copy(
                    zeros_v, out_hbm.at[dstz_v.at[b]], sem_s),
                pltpu.make_async_copy(
                    rows_v.at[b], out_hbm.at[dst_v.at[b]], sem_s),
            )

        gather_of(0, 0).start()

        def half_chunk(i2, b):
            i = 2 * i2 + b
            gather_of(i, b).wait()

            def build(k, _):
                sl = pl.ds(k * 16, 16)
                rv = rol_v[pl.ds(i * C + k * 16, 16)]
                p = (2 * (base + i * C + k * 16)
                     + 2 * lax.iota(jnp.int32, 16))
                dst_v[b, sl] = p + rv
                dstz_v[b, sl] = p + (1 - rv)
                return _

            lax.fori_loop(0, C // 16, build, 0)

            @pl.when(i >= 1)
            def _():
                s0, s1 = scatters_of(i, b)
                s0.wait()
                s1.wait()

            @pl.when(i + 1 < nch)
            def _():
                gather_of(i + 1, 1 - b).start()

            s0, s1 = scatters_of(i, b)
            s0.start()
            s1.start()

        def loop_body(i2, carry):
            half_chunk(i2, 0)
            half_chunk(i2, 1)
            return carry

        lax.fori_loop(0, nch // 2, loop_body, 0)
        s0, s1 = scatters_of(nch - 1, 1)
        s0.wait()
        s1.wait()

    return gather_kernel(ids, role, table)


def _tc_matmul(xa, wstack, blk):
    N = xa.shape[0]
    D = wstack.shape[1]

    def body(x_ref, w_ref, o_ref):
        o_ref[...] = jnp.dot(
            x_ref[...], w_ref[...], preferred_element_type=jnp.float32
        )

    return pl.pallas_call(
        body,
        grid=(N // blk,),
        in_specs=[
            pl.BlockSpec((blk, 2 * D), lambda i: (i, 0)),
            pl.BlockSpec((2 * D, D), lambda i: (0, 0)),
        ],
        out_specs=pl.BlockSpec((blk, D), lambda i: (i, 0)),
        out_shape=jax.ShapeDtypeStruct((N, D), jnp.float32),
    )(xa, wstack)


def kernel(input_ids, role_mask, table, W0, W1):
    B, L = input_ids.shape
    V, D = table.shape
    N = B * L
    ids = input_ids.reshape(N).astype(jnp.int32)
    role = role_mask.reshape(N).astype(jnp.int32)
    xa2 = _sc_gather_pairs(ids, role, table, V, D)
    xa = xa2.reshape(N, 2 * D)
    wstack = jnp.concatenate([W0.T, W1.T], axis=0)
    out = _tc_matmul(xa, wstack, blk=4096)
    return out.reshape(B, L, D)


# speedup vs baseline: 13.4353x; 1.0282x over previous
import functools

import jax
import jax.numpy as jnp
from jax import lax
from jax.experimental import pallas as pl
from jax.experimental.pallas import tpu as pltpu
from jax.experimental.pallas import tpu_sc as plsc


def _sc_gather_pairs(ids, role, table, V, D):
    N = ids.shape[0]
    info = plsc.get_sparse_core_info()
    NC, NS = info.num_cores, info.num_subcores
    NW = NC * NS
    per_w = N // NW
    C = 320
    nch = per_w // C
    assert per_w % C == 0 and N % NW == 0 and nch % 2 == 0

    mesh = plsc.VectorSubcoreMesh(core_axis_name="c", subcore_axis_name="s")

    @functools.partial(
        pl.kernel,
        mesh=mesh,
        out_type=jax.ShapeDtypeStruct((2 * N, D), jnp.float32),
        scratch_types=[
            pltpu.VMEM((per_w,), jnp.int32),
            pltpu.VMEM((per_w,), jnp.int32),
            pltpu.VMEM((2, C), jnp.int32),
            pltpu.VMEM((2, C), jnp.int32),
            pltpu.VMEM((C, D), jnp.float32),
            pltpu.VMEM((2, C, D), jnp.float32),
            pltpu.SemaphoreType.DMA,
            pltpu.SemaphoreType.DMA,
        ],
        compiler_params=pltpu.CompilerParams(
            use_tc_tiling_on_sc=False, needs_layout_passes=False
        ),
    )
    def gather_kernel(ids_hbm, role_hbm, tbl_hbm, out_hbm,
                      ids_v, rol_v, dst_v, dstz_v, zeros_v, rows_v,
                      sem_g, sem_s):
        sid = lax.axis_index("s")
        wid = sid * NC + lax.axis_index("c")
        base = wid * per_w

        pltpu.sync_copy(ids_hbm.at[pl.ds(base, per_w)], ids_v)
        pltpu.sync_copy(role_hbm.at[pl.ds(base, per_w)], rol_v)

        def zinit(j, _):
            z = jnp.zeros((16,), jnp.float32)
            zeros_v[j, pl.ds(0, 16)] = z
            zeros_v[j, pl.ds(16, 16)] = z
            zeros_v[j, pl.ds(32, 16)] = z
            zeros_v[j, pl.ds(48, 16)] = z
            return _

        lax.fori_loop(0, C, zinit, 0)

        def gather_of(i, b):
            return pltpu.make_async_copy(
                tbl_hbm.at[ids_v.at[pl.ds(i * C, C)]],
                rows_v.at[b], sem_g,
            )

        def scatters_of(i, b):
            return (
                pltpu.make_async_copy(
                    zeros_v, out_hbm.at[dstz_v.at[b]], sem_s),
                pltpu.make_async_copy(
                    rows_v.at[b], out_hbm.at[dst_v.at[b]], sem_s),
            )

        gather_of(0, 0).start()

        def half_chunk(i2, b):
            i = 2 * i2 + b
            gather_of(i, b).wait()

            def build(k, _):
                sl = pl.ds(k * 16, 16)
                rv = rol_v[pl.ds(i * C + k * 16, 16)]
                p = (2 * (base + i * C + k * 16)
                     + 2 * lax.iota(jnp.int32, 16))
                dst_v[b, sl] = p + rv
                dstz_v[b, sl] = p + (1 - rv)
                return _

            lax.fori_loop(0, C // 16, build, 0)

            @pl.when(i >= 1)
            def _():
                s0, s1 = scatters_of(i, b)
                s0.wait()
                s1.wait()

            @pl.when(i + 1 < nch)
            def _():
                gather_of(i + 1, 1 - b).start()

            s0, s1 = scatters_of(i, b)
            s0.start()
            s1.start()

        def loop_body(i2, carry):
            half_chunk(i2, 0)
            half_chunk(i2, 1)
            return carry

        lax.fori_loop(0, nch // 2, loop_body, 0)
        s0, s1 = scatters_of(nch - 1, 1)
        s0.wait()
        s1.wait()

    return gather_kernel(ids, role, table)


def _tc_matmul(xa, wstack, blk):
    N = xa.shape[0]
    D = wstack.shape[1]

    def body(x_ref, w_ref, o_ref):
        o_ref[...] = jnp.dot(
            x_ref[...], w_ref[...], preferred_element_type=jnp.float32
        )

    return pl.pallas_call(
        body,
        grid=(N // blk,),
        in_specs=[
            pl.BlockSpec((blk, 2 * D), lambda i: (i, 0)),
            pl.BlockSpec((2 * D, D), lambda i: (0, 0)),
        ],
        out_specs=pl.BlockSpec((blk, D), lambda i: (i, 0)),
        out_shape=jax.ShapeDtypeStruct((N, D), jnp.float32),
    )(xa, wstack)


def kernel(input_ids, role_mask, table, W0, W1):
    B, L = input_ids.shape
    V, D = table.shape
    N = B * L
    ids = input_ids.reshape(N).astype(jnp.int32)
    role = role_mask.reshape(N).astype(jnp.int32)
    xa2 = _sc_gather_pairs(ids, role, table, V, D)
    xa = xa2.reshape(N, 2 * D)
    wstack = jnp.concatenate([W0.T, W1.T], axis=0)
    out = _tc_matmul(xa, wstack, blk=8192)
    return out.reshape(B, L, D)


# speedup vs baseline: 13.9522x; 1.0385x over previous
import functools

import jax
import jax.numpy as jnp
from jax import lax
from jax.experimental import pallas as pl
from jax.experimental.pallas import tpu as pltpu
from jax.experimental.pallas import tpu_sc as plsc


def _sc_gather_pairs(ids, role, table, V, D):
    N = ids.shape[0]
    info = plsc.get_sparse_core_info()
    NC, NS = info.num_cores, info.num_subcores
    NW = NC * NS
    per_w = N // NW
    C = 320
    nch = per_w // C
    assert per_w % C == 0 and N % NW == 0 and nch % 2 == 0

    mesh = plsc.VectorSubcoreMesh(core_axis_name="c", subcore_axis_name="s")

    @functools.partial(
        pl.kernel,
        mesh=mesh,
        out_type=jax.ShapeDtypeStruct((2 * N, D), jnp.float32),
        scratch_types=[
            pltpu.VMEM((per_w,), jnp.int32),
            pltpu.VMEM((per_w,), jnp.int32),
            pltpu.VMEM((2, C), jnp.int32),
            pltpu.VMEM((2, C), jnp.int32),
            pltpu.VMEM((C, D), jnp.float32),
            pltpu.VMEM((2, C, D), jnp.float32),
            pltpu.SemaphoreType.DMA,
            pltpu.SemaphoreType.DMA,
        ],
        compiler_params=pltpu.CompilerParams(
            use_tc_tiling_on_sc=False, needs_layout_passes=False
        ),
    )
    def gather_kernel(ids_hbm, role_hbm, tbl_hbm, out_hbm,
                      ids_v, rol_v, dst_v, dstz_v, zeros_v, rows_v,
                      sem_g, sem_s):
        sid = lax.axis_index("s")
        wid = sid * NC + lax.axis_index("c")
        base = wid * per_w

        pltpu.sync_copy(ids_hbm.at[pl.ds(base, per_w)], ids_v)
        pltpu.sync_copy(role_hbm.at[pl.ds(base, per_w)], rol_v)

        def zinit(j, _):
            z = jnp.zeros((16,), jnp.float32)
            zeros_v[j, pl.ds(0, 16)] = z
            zeros_v[j, pl.ds(16, 16)] = z
            zeros_v[j, pl.ds(32, 16)] = z
            zeros_v[j, pl.ds(48, 16)] = z
            return _

        lax.fori_loop(0, C, zinit, 0)

        def gather_of(i, b):
            return pltpu.make_async_copy(
                tbl_hbm.at[ids_v.at[pl.ds(i * C, C)]],
                rows_v.at[b], sem_g,
            )

        def scatters_of(i, b):
            return (
                pltpu.make_async_copy(
                    zeros_v, out_hbm.at[dstz_v.at[b]], sem_s),
                pltpu.make_async_copy(
                    rows_v.at[b], out_hbm.at[dst_v.at[b]], sem_s),
            )

        gather_of(0, 0).start()

        def half_chunk(i2, b):
            i = 2 * i2 + b
            gather_of(i, b).wait()

            def build(k, _):
                sl = pl.ds(k * 16, 16)
                rv = rol_v[pl.ds(i * C + k * 16, 16)]
                p = (2 * (base + i * C + k * 16)
                     + 2 * lax.iota(jnp.int32, 16))
                dst_v[b, sl] = p + rv
                dstz_v[b, sl] = p + (1 - rv)
                return _

            lax.fori_loop(0, C // 16, build, 0)

            @pl.when(i >= 1)
            def _():
                s0, s1 = scatters_of(i, b)
                s0.wait()
                s1.wait()

            @pl.when(i + 1 < nch)
            def _():
                gather_of(i + 1, 1 - b).start()

            s0, s1 = scatters_of(i, b)
            s0.start()
            s1.start()

        def loop_body(i2, carry):
            half_chunk(i2, 0)
            half_chunk(i2, 1)
            return carry

        lax.fori_loop(0, nch // 2, loop_body, 0)
        s0, s1 = scatters_of(nch - 1, 1)
        s0.wait()
        s1.wait()

    return gather_kernel(ids, role, table)


def _tc_transpose_pack(tableT, V, D, blk):
    def body(t_ref, o_ref):
        y = jnp.transpose(t_ref[...])
        y3 = y.reshape(blk // 2, 2, D)
        o_ref[...] = jnp.concatenate([y3[:, 0, :], y3[:, 1, :]], axis=1)

    return pl.pallas_call(
        body,
        grid=((V + blk - 1) // blk,),
        in_specs=[pl.BlockSpec((D, blk), lambda i: (0, i))],
        out_specs=pl.BlockSpec((blk // 2, 2 * D), lambda i: (i, 0)),
        out_shape=jax.ShapeDtypeStruct((V // 2, 2 * D), jnp.float32),
    )(tableT)


def _tc_matmul(xa, wstack, blk):
    N = xa.shape[0]
    D = wstack.shape[1]

    def body(x_ref, w_ref, o_ref):
        o_ref[...] = jnp.dot(
            x_ref[...], w_ref[...], preferred_element_type=jnp.float32
        )

    return pl.pallas_call(
        body,
        grid=(N // blk,),
        in_specs=[
            pl.BlockSpec((blk, 2 * D), lambda i: (i, 0)),
            pl.BlockSpec((2 * D, D), lambda i: (0, 0)),
        ],
        out_specs=pl.BlockSpec((blk, D), lambda i: (i, 0)),
        out_shape=jax.ShapeDtypeStruct((N, D), jnp.float32),
    )(xa, wstack)


def kernel(input_ids, role_mask, table, W0, W1):
    B, L = input_ids.shape
    V, D = table.shape
    N = B * L
    ids = input_ids.reshape(N).astype(jnp.int32)
    role = role_mask.reshape(N).astype(jnp.int32)
    tbl = _tc_transpose_pack(table.T, V, D, blk=2048).reshape(V, D)
    xa2 = _sc_gather_pairs(ids, role, tbl, V, D)
    xa = xa2.reshape(N, 2 * D)
    wstack = jnp.concatenate([W0.T, W1.T], axis=0)
    out = _tc_matmul(xa, wstack, blk=8192)
    return out.reshape(B, L, D)


# speedup vs baseline: 15.6808x; 1.1239x over previous
import functools

import jax
import jax.numpy as jnp
from jax import lax
from jax.experimental import pallas as pl
from jax.experimental.pallas import tpu as pltpu
from jax.experimental.pallas import tpu_sc as plsc


def _sc_gather_pairs(ids, role, table, V, D):
    N = ids.shape[0]
    info = plsc.get_sparse_core_info()
    NC, NS = info.num_cores, info.num_subcores
    NW = NC * NS
    per_w = N // NW
    C = 320
    nch = per_w // C
    assert per_w % C == 0 and N % NW == 0 and nch % 2 == 0

    mesh = plsc.VectorSubcoreMesh(core_axis_name="c", subcore_axis_name="s")

    @functools.partial(
        pl.kernel,
        mesh=mesh,
        out_type=jax.ShapeDtypeStruct((2 * N, D), jnp.float32),
        scratch_types=[
            pltpu.VMEM((per_w,), jnp.int32),
            pltpu.VMEM((per_w,), jnp.int32),
            pltpu.VMEM((2, C), jnp.int32),
            pltpu.VMEM((2, C), jnp.int32),
            pltpu.VMEM((C, D), jnp.float32),
            pltpu.VMEM((2, C, D), jnp.float32),
            pltpu.SemaphoreType.DMA,
            pltpu.SemaphoreType.DMA,
        ],
        compiler_params=pltpu.CompilerParams(
            use_tc_tiling_on_sc=False, needs_layout_passes=False
        ),
    )
    def gather_kernel(ids_hbm, role_hbm, tbl_hbm, out_hbm,
                      ids_v, rol_v, dst_v, dstz_v, zeros_v, rows_v,
                      sem_g, sem_s):
        sid = lax.axis_index("s")
        wid = sid * NC + lax.axis_index("c")
        base = wid * per_w

        pltpu.sync_copy(ids_hbm.at[pl.ds(base, per_w)], ids_v)
        pltpu.sync_copy(role_hbm.at[pl.ds(base, per_w)], rol_v)

        def zinit(j, _):
            z = jnp.zeros((16,), jnp.float32)
            zeros_v[j, pl.ds(0, 16)] = z
            zeros_v[j, pl.ds(16, 16)] = z
            zeros_v[j, pl.ds(32, 16)] = z
            zeros_v[j, pl.ds(48, 16)] = z
            return _

        lax.fori_loop(0, C, zinit, 0)

        def gather_of(i, b):
            return pltpu.make_async_copy(
                tbl_hbm.at[ids_v.at[pl.ds(i * C, C)]],
                rows_v.at[b], sem_g,
            )

        def scatters_of(i, b):
            return (
                pltpu.make_async_copy(
                    zeros_v, out_hbm.at[dstz_v.at[b]], sem_s),
                pltpu.make_async_copy(
                    rows_v.at[b], out_hbm.at[dst_v.at[b]], sem_s),
            )

        gather_of(0, 0).start()

        def half_chunk(i2, b):
            i = 2 * i2 + b
            gather_of(i, b).wait()

            def build(k, _):
                sl = pl.ds(k * 16, 16)
                rv = rol_v[pl.ds(i * C + k * 16, 16)]
                p = (2 * (base + i * C + k * 16)
                     + 2 * lax.iota(jnp.int32, 16))
                dst_v[b, sl] = p + rv
                dstz_v[b, sl] = p + (1 - rv)
                return _

            lax.fori_loop(0, C // 16, build, 0)

            @pl.when(i >= 1)
            def _():
                s0, s1 = scatters_of(i, b)
                s0.wait()
                s1.wait()

            @pl.when(i + 1 < nch)
            def _():
                gather_of(i + 1, 1 - b).start()

            s0, s1 = scatters_of(i, b)
            s0.start()
            s1.start()

        def loop_body(i2, carry):
            half_chunk(i2, 0)
            half_chunk(i2, 1)
            return carry

        lax.fori_loop(0, nch // 2, loop_body, 0)
        s0, s1 = scatters_of(nch - 1, 1)
        s0.wait()
        s1.wait()

    return gather_kernel(ids, role, table)


def _tc_transpose_pack(tableT, V, D, blk):
    def body(t_ref, o_ref):
        y = jnp.transpose(t_ref[...])
        y3 = y.reshape(blk // 2, 2, D)
        o_ref[...] = jnp.concatenate([y3[:, 0, :], y3[:, 1, :]], axis=1)

    return pl.pallas_call(
        body,
        grid=((V + blk - 1) // blk,),
        in_specs=[pl.BlockSpec((D, blk), lambda i: (0, i))],
        out_specs=pl.BlockSpec((blk // 2, 2 * D), lambda i: (i, 0)),
        out_shape=jax.ShapeDtypeStruct((V // 2, 2 * D), jnp.float32),
    )(tableT)


def _tc_matmul(xa, wstack, blk):
    N = xa.shape[0]
    D = wstack.shape[1]

    def body(x_ref, w_ref, o_ref):
        o_ref[...] = jnp.dot(
            x_ref[...], w_ref[...], preferred_element_type=jnp.float32
        )

    return pl.pallas_call(
        body,
        grid=(N // blk,),
        in_specs=[
            pl.BlockSpec((blk, 2 * D), lambda i: (i, 0)),
            pl.BlockSpec((2 * D, D), lambda i: (0, 0)),
        ],
        out_specs=pl.BlockSpec((blk, D), lambda i: (i, 0)),
        out_shape=jax.ShapeDtypeStruct((N, D), jnp.float32),
    )(xa, wstack)


def kernel(input_ids, role_mask, table, W0, W1):
    B, L = input_ids.shape
    V, D = table.shape
    N = B * L
    ids = input_ids.reshape(N).astype(jnp.int32)
    role = role_mask.reshape(N).astype(jnp.int32)
    tbl = _tc_transpose_pack(table.T, V, D, blk=8192).reshape(V, D)
    xa2 = _sc_gather_pairs(ids, role, tbl, V, D)
    xa = xa2.reshape(N, 2 * D)
    wstack = jnp.concatenate([W0.T, W1.T], axis=0)
    out = _tc_matmul(xa, wstack, blk=8192)
    return out.reshape(B, L, D)


# speedup vs baseline: 15.7521x; 1.0045x over previous
import functools

import jax
import jax.numpy as jnp
from jax import lax
from jax.experimental import pallas as pl
from jax.experimental.pallas import tpu as pltpu
from jax.experimental.pallas import tpu_sc as plsc


def _sc_gather_pairs(ids, role, table, V, D):
    N = ids.shape[0]
    info = plsc.get_sparse_core_info()
    NC, NS = info.num_cores, info.num_subcores
    NW = NC * NS
    per_w = N // NW
    C = 320
    nch = per_w // C
    assert per_w % C == 0 and N % NW == 0 and nch % 2 == 0

    mesh = plsc.VectorSubcoreMesh(core_axis_name="c", subcore_axis_name="s")

    @functools.partial(
        pl.kernel,
        mesh=mesh,
        out_type=jax.ShapeDtypeStruct((2 * N, D), jnp.float32),
        scratch_types=[
            pltpu.VMEM((per_w,), jnp.int32),
            pltpu.VMEM((per_w,), jnp.int32),
            pltpu.VMEM((2, C), jnp.int32),
            pltpu.VMEM((2, C), jnp.int32),
            pltpu.VMEM((C, D), jnp.float32),
            pltpu.VMEM((2, C, D), jnp.float32),
            pltpu.SemaphoreType.DMA,
            pltpu.SemaphoreType.DMA,
        ],
        compiler_params=pltpu.CompilerParams(
            use_tc_tiling_on_sc=False, needs_layout_passes=False
        ),
    )
    def gather_kernel(ids_hbm, role_hbm, tbl_hbm, out_hbm,
                      ids_v, rol_v, dst_v, dstz_v, zeros_v, rows_v,
                      sem_g, sem_s):
        sid = lax.axis_index("s")
        wid = sid * NC + lax.axis_index("c")
        base = wid * per_w

        pltpu.sync_copy(ids_hbm.at[pl.ds(base, per_w)], ids_v)
        pltpu.sync_copy(role_hbm.at[pl.ds(base, per_w)], rol_v)

        def zinit(j, _):
            z = jnp.zeros((16,), jnp.float32)
            zeros_v[j, pl.ds(0, 16)] = z
            zeros_v[j, pl.ds(16, 16)] = z
            zeros_v[j, pl.ds(32, 16)] = z
            zeros_v[j, pl.ds(48, 16)] = z
            return _

        lax.fori_loop(0, C, zinit, 0)

        def gather_of(i, b):
            return pltpu.make_async_copy(
                tbl_hbm.at[ids_v.at[pl.ds(i * C, C)]],
                rows_v.at[b], sem_g,
            )

        def scatters_of(i, b):
            return (
                pltpu.make_async_copy(
                    zeros_v, out_hbm.at[dstz_v.at[b]], sem_s),
                pltpu.make_async_copy(
                    rows_v.at[b], out_hbm.at[dst_v.at[b]], sem_s),
            )

        gather_of(0, 0).start()

        def half_chunk(i2, b):
            i = 2 * i2 + b
            gather_of(i, b).wait()

            def build(k, _):
                sl = pl.ds(k * 16, 16)
                rv = rol_v[pl.ds(i * C + k * 16, 16)]
                p = (2 * (base + i * C + k * 16)
                     + 2 * lax.iota(jnp.int32, 16))
                dst_v[b, sl] = p + rv
                dstz_v[b, sl] = p + (1 - rv)
                return _

            lax.fori_loop(0, C // 16, build, 0)

            @pl.when(i >= 1)
            def _():
                s0, s1 = scatters_of(i, b)
                s0.wait()
                s1.wait()

            @pl.when(i + 1 < nch)
            def _():
                gather_of(i + 1, 1 - b).start()

            s0, s1 = scatters_of(i, b)
            s0.start()
            s1.start()

        def loop_body(i2, carry):
            half_chunk(i2, 0)
            half_chunk(i2, 1)
            return carry

        lax.fori_loop(0, nch // 2, loop_body, 0)
        s0, s1 = scatters_of(nch - 1, 1)
        s0.wait()
        s1.wait()

    return gather_kernel(ids, role, table)


def _tc_transpose_pack(tableT, V, D, blk):
    def body(t_ref, o_ref):
        y = jnp.transpose(t_ref[...])
        y3 = y.reshape(blk // 2, 2, D)
        o_ref[...] = jnp.concatenate([y3[:, 0, :], y3[:, 1, :]], axis=1)

    return pl.pallas_call(
        body,
        grid=((V + blk - 1) // blk,),
        in_specs=[pl.BlockSpec((D, blk), lambda i: (0, i))],
        out_specs=pl.BlockSpec((blk // 2, 2 * D), lambda i: (i, 0)),
        out_shape=jax.ShapeDtypeStruct((V // 2, 2 * D), jnp.float32),
    )(tableT)


def _tc_matmul(xa, wstack, blk):
    N = xa.shape[0]
    D = wstack.shape[1]

    def body(x_ref, w_ref, o_ref):
        o_ref[...] = jnp.dot(
            x_ref[...], w_ref[...], preferred_element_type=jnp.float32
        )

    return pl.pallas_call(
        body,
        grid=(N // blk,),
        in_specs=[
            pl.BlockSpec((blk, 2 * D), lambda i: (i, 0)),
            pl.BlockSpec((2 * D, D), lambda i: (0, 0)),
        ],
        out_specs=pl.BlockSpec((blk, D), lambda i: (i, 0)),
        out_shape=jax.ShapeDtypeStruct((N, D), jnp.float32),
    )(xa, wstack)


def kernel(input_ids, role_mask, table, W0, W1):
    B, L = input_ids.shape
    V, D = table.shape
    N = B * L
    ids = input_ids.reshape(N).astype(jnp.int32)
    role = role_mask.reshape(N).astype(jnp.int32)
    tbl = _tc_transpose_pack(table.T, V, D, blk=16384).reshape(V, D)
    xa2 = _sc_gather_pairs(ids, role, tbl, V, D)
    xa = xa2.reshape(N, 2 * D)
    wstack = jnp.concatenate([W0.T, W1.T], axis=0)
    out = _tc_matmul(xa, wstack, blk=8192)
    return out.reshape(B, L, D)
